# baseline (speedup 1.0000x reference)
import functools

import jax
import jax.numpy as jnp
from jax import lax
from jax.experimental import pallas as pl
from jax.experimental.pallas import tpu as pltpu
from jax.experimental.pallas import tpu_sc as plsc

_SUB = 2
_L = 16
_NC = 2
_NS = 16
_NW = _NC * _NS


def _tc_coord_body(x_ref, o_ref, m_ref):
    h, blk = m_ref.shape

    @pl.when(pl.program_id(0) == 0)
    def _():
        i = jax.lax.broadcasted_iota(jnp.int32, (h, blk), 0)
        j = jax.lax.broadcasted_iota(jnp.int32, (h, blk), 1)
        sel = (j == 2 * i) | (j == 2 * i + 1)
        m_ref[...] = jnp.where(sel, jnp.float32(0.5), jnp.float32(0.0))

    o_ref[...] = jax.lax.dot_general(
        m_ref[...], x_ref[...], (((1,), (0,)), ((), ())),
        preferred_element_type=jnp.float32)


def tc_coords(coords):
    N = coords.shape[0]
    BLK = 256
    return pl.pallas_call(
        _tc_coord_body,
        grid=(N // BLK,),
        in_specs=[pl.BlockSpec((BLK, 3), lambda i: (i, 0))],
        out_specs=pl.BlockSpec((BLK // 2, 3), lambda i: (i, 0)),
        out_shape=jax.ShapeDtypeStruct((N // 2, 3), jnp.float32),
        scratch_shapes=[pltpu.VMEM((BLK // 2, BLK), jnp.float32)],
    )(coords)


def kernel(coords, input_feat, seps):
    N, D = input_feat.shape
    R = N // _SUB
    B = seps.shape[0]
    assert N % (_SUB * _NW) == 0 and D % _L == 0 and B == _L

    in_rows = N // _NW
    FC = 176
    n_chunks = 2 * (-(-in_rows // (2 * FC)))
    max_start = in_rows - FC
    assert max_start % 16 == 0

    out_rows = R // _NW
    GPT = 313
    CROWS = GPT * 16
    CIN = GPT * 96
    CWORDS = GPT * 48
    assert (R - CROWS) % 8 == 0

    upt = N // _NW
    uvecs = upt // _L

    mesh = plsc.VectorSubcoreMesh(
        core_axis_name="c", subcore_axis_name="s",
        num_cores=_NC, num_subcores=_NS)

    @functools.partial(
        pl.kernel,
        out_type=[
            jax.ShapeDtypeStruct((R, D), jnp.float32),
            jax.ShapeDtypeStruct((B,), jnp.int32),
            jax.ShapeDtypeStruct((N,), jnp.int32),
        ],
        mesh=mesh,
        scratch_types=[
            pltpu.VMEM((FC, D), jnp.float32),
            pltpu.VMEM((FC, D), jnp.float32),
            pltpu.VMEM((FC // 2, D), jnp.float32),
            pltpu.VMEM((FC // 2, D), jnp.float32),
            pltpu.VMEM((upt,), jnp.int32),
            pltpu.VMEM((_L,), jnp.int32),
            pltpu.SemaphoreType.DMA,
            pltpu.SemaphoreType.DMA,
            pltpu.SemaphoreType.DMA,
            pltpu.SemaphoreType.DMA,
        ],
    )
    def sc_kernel(feat_hbm, seps_hbm,
                  out_feat, out_sep, out_unpool,
                  fin0, fin1, fout0, fout1, ubuf, sbuf,
                  si0, si1, so0, so1):
        wid = lax.axis_index("s") * _NC + lax.axis_index("c")
        iota = lax.iota(jnp.int32, _L)

        row0 = wid * in_rows
        n_pairs = n_chunks // 2

        def r_in(g):
            r = row0 + jnp.minimum(g * FC, max_start)
            return pl.multiple_of(r, 16)

        def in_slice(g):
            return feat_hbm.at[pl.ds(r_in(g), FC), :]

        def out_slice(g):
            r2 = lax.shift_right_logical(r_in(g), 1)
            return out_feat.at[pl.ds(pl.multiple_of(r2, 8), FC // 2), :]

        def compute(fin, fout):
            @pl.loop(0, FC // 2)
            def _row(r):
                for q in range(D // _L):
                    a = fin[2 * r, pl.ds(q * _L, _L)]
                    b = fin[2 * r + 1, pl.ds(q * _L, _L)]
                    fout[r, pl.ds(q * _L, _L)] = jnp.maximum(a, b)

        pltpu.async_copy(in_slice(0), fin0, si0)
        pltpu.async_copy(in_slice(1), fin1, si1)

        @pl.loop(0, n_pairs)
        def _feat(gg):
            g0 = gg * 2
            for (b, fin, fout, si, so) in ((0, fin0, fout0, si0, so0),
                                           (1, fin1, fout1, si1, so1)):
                @pl.when(gg > 0)
                def _():
                    pltpu.make_async_copy(fout, out_slice(b), so).wait()

                pltpu.make_async_copy(in_slice(b), fin, si).wait()
                compute(fin, fout)
                pltpu.async_copy(fout, out_slice(g0 + b), so)

                @pl.when(gg + 1 < n_pairs)
                def _():
                    pltpu.async_copy(in_slice(g0 + b + 2), fin, si)

        pltpu.make_async_copy(fout0, out_slice(0), so0).wait()
        pltpu.make_async_copy(fout1, out_slice(1), so1).wait()

        half = lax.shift_right_logical(iota, 1)
        u0 = wid * (upt // 2)

        @pl.loop(0, uvecs)
        def _unpool(v):
            ubuf[pl.ds(v * _L, _L)] = u0 + v * (_L // 2) + half

        pltpu.sync_copy(ubuf, out_unpool.at[pl.ds(wid * upt, upt)])

        @pl.when(wid == 0)
        def _sep():
            pltpu.sync_copy(seps_hbm, sbuf)
            sbuf[...] = lax.shift_right_logical(sbuf[...] + 1, 1)
            pltpu.sync_copy(sbuf, out_sep)

    rf, rs, ui = sc_kernel(input_feat, seps)
    rc = tc_coords(coords)
    return rf, rc, rs, ui


# speedup vs baseline: 3.0017x; 3.0017x over previous
import functools

import jax
import jax.numpy as jnp
from jax import lax
from jax.experimental import pallas as pl
from jax.experimental.pallas import tpu as pltpu
from jax.experimental.pallas import tpu_sc as plsc

_SUB = 2
_L = 16
_NC = 2
_NS = 16
_NW = _NC * _NS


def _tc_coord_body(x_ref, o_ref, m_ref):
    h, sub = m_ref.shape

    @pl.when(pl.program_id(0) == 0)
    def _():
        i = jax.lax.broadcasted_iota(jnp.int32, (h, sub), 0)
        j = jax.lax.broadcasted_iota(jnp.int32, (h, sub), 1)
        sel = (j == 2 * i) | (j == 2 * i + 1)
        m_ref[...] = jnp.where(sel, jnp.float32(0.5), jnp.float32(0.0))

    for t in range(x_ref.shape[0] // sub):
        o_ref[pl.ds(t * h, h), :] = jax.lax.dot_general(
            m_ref[...], x_ref[pl.ds(t * sub, sub), :],
            (((1,), (0,)), ((), ())),
            preferred_element_type=jnp.float32)


def tc_coords(coords):
    N = coords.shape[0]
    BLK = 3200
    return pl.pallas_call(
        _tc_coord_body,
        grid=(N // BLK,),
        in_specs=[pl.BlockSpec((BLK, 3), lambda i: (i, 0))],
        out_specs=pl.BlockSpec((BLK // 2, 3), lambda i: (i, 0)),
        out_shape=jax.ShapeDtypeStruct((N // 2, 3), jnp.float32),
        scratch_shapes=[pltpu.VMEM((64, 128), jnp.float32)],
    )(coords)


def kernel(coords, input_feat, seps):
    N, D = input_feat.shape
    R = N // _SUB
    B = seps.shape[0]
    assert N % (_SUB * _NW) == 0 and D % _L == 0 and B == _L

    in_rows = N // _NW
    FC = 176
    n_chunks = 2 * (-(-in_rows // (2 * FC)))
    max_start = in_rows - FC
    assert max_start % 16 == 0

    out_rows = R // _NW
    GPT = 313
    CROWS = GPT * 16
    CIN = GPT * 96
    CWORDS = GPT * 48
    assert (R - CROWS) % 8 == 0

    upt = N // _NW
    uvecs = upt // _L

    mesh = plsc.VectorSubcoreMesh(
        core_axis_name="c", subcore_axis_name="s",
        num_cores=_NC, num_subcores=_NS)

    @functools.partial(
        pl.kernel,
        out_type=[
            jax.ShapeDtypeStruct((R, D), jnp.float32),
            jax.ShapeDtypeStruct((B,), jnp.int32),
            jax.ShapeDtypeStruct((N,), jnp.int32),
        ],
        mesh=mesh,
        scratch_types=[
            pltpu.VMEM((FC, D), jnp.float32),
            pltpu.VMEM((FC, D), jnp.float32),
            pltpu.VMEM((FC // 2, D), jnp.float32),
            pltpu.VMEM((FC // 2, D), jnp.float32),
            pltpu.VMEM((upt,), jnp.int32),
            pltpu.VMEM((_L,), jnp.int32),
            pltpu.SemaphoreType.DMA,
            pltpu.SemaphoreType.DMA,
            pltpu.SemaphoreType.DMA,
            pltpu.SemaphoreType.DMA,
        ],
    )
    def sc_kernel(feat_hbm, seps_hbm,
                  out_feat, out_sep, out_unpool,
                  fin0, fin1, fout0, fout1, ubuf, sbuf,
                  si0, si1, so0, so1):
        wid = lax.axis_index("s") * _NC + lax.axis_index("c")
        iota = lax.iota(jnp.int32, _L)

        row0 = wid * in_rows
        n_pairs = n_chunks // 2

        def r_in(g):
            r = row0 + jnp.minimum(g * FC, max_start)
            return pl.multiple_of(r, 16)

        def in_slice(g):
            return feat_hbm.at[pl.ds(r_in(g), FC), :]

        def out_slice(g):
            r2 = lax.shift_right_logical(r_in(g), 1)
            return out_feat.at[pl.ds(pl.multiple_of(r2, 8), FC // 2), :]

        def compute(fin, fout):
            @pl.loop(0, FC // 2)
            def _row(r):
                for q in range(D // _L):
                    a = fin[2 * r, pl.ds(q * _L, _L)]
                    b = fin[2 * r + 1, pl.ds(q * _L, _L)]
                    fout[r, pl.ds(q * _L, _L)] = jnp.maximum(a, b)

        pltpu.async_copy(in_slice(0), fin0, si0)
        pltpu.async_copy(in_slice(1), fin1, si1)

        @pl.loop(0, n_pairs)
        def _feat(gg):
            g0 = gg * 2
            for (b, fin, fout, si, so) in ((0, fin0, fout0, si0, so0),
                                           (1, fin1, fout1, si1, so1)):
                @pl.when(gg > 0)
                def _():
                    pltpu.make_async_copy(fout, out_slice(b), so).wait()

                pltpu.make_async_copy(in_slice(b), fin, si).wait()
                compute(fin, fout)
                pltpu.async_copy(fout, out_slice(g0 + b), so)

                @pl.when(gg + 1 < n_pairs)
                def _():
                    pltpu.async_copy(in_slice(g0 + b + 2), fin, si)

        pltpu.make_async_copy(fout0, out_slice(0), so0).wait()
        pltpu.make_async_copy(fout1, out_slice(1), so1).wait()

        half = lax.shift_right_logical(iota, 1)
        u0 = wid * (upt // 2)

        @pl.loop(0, uvecs)
        def _unpool(v):
            ubuf[pl.ds(v * _L, _L)] = u0 + v * (_L // 2) + half

        pltpu.sync_copy(ubuf, out_unpool.at[pl.ds(wid * upt, upt)])

        @pl.when(wid == 0)
        def _sep():
            pltpu.sync_copy(seps_hbm, sbuf)
            sbuf[...] = lax.shift_right_logical(sbuf[...] + 1, 1)
            pltpu.sync_copy(sbuf, out_sep)

    rf, rs, ui = sc_kernel(input_feat, seps)
    rc = tc_coords(coords)
    return rf, rc, rs, ui


# speedup vs baseline: 3.8987x; 1.2988x over previous
import functools

import jax
import jax.numpy as jnp
from jax import lax
from jax.experimental import pallas as pl
from jax.experimental.pallas import tpu as pltpu
from jax.experimental.pallas import tpu_sc as plsc

_SUB = 2
_L = 16
_NC = 2
_NS = 16
_NW = _NC * _NS


def kernel(coords, input_feat, seps):
    N, D = input_feat.shape
    R = N // _SUB
    B = seps.shape[0]
    assert N % (_SUB * _NW) == 0 and D % _L == 0 and B == _L

    in_rows = N // _NW
    FC = 128
    n_chunks = 2 * (-(-in_rows // (2 * FC)))
    max_start = in_rows - FC
    assert max_start % 16 == 0

    out_rows = R // _NW
    CROWS = 5120
    assert CROWS % 128 == 0 and (R - CROWS) % 128 == 0

    upt = N // _NW
    uvecs = upt // _L

    mesh = plsc.VectorSubcoreMesh(
        core_axis_name="c", subcore_axis_name="s",
        num_cores=_NC, num_subcores=_NS)

    @functools.partial(
        pl.kernel,
        out_type=[
            jax.ShapeDtypeStruct((R, D), jnp.float32),
            jax.ShapeDtypeStruct((3, R), jnp.float32),
            jax.ShapeDtypeStruct((B,), jnp.int32),
            jax.ShapeDtypeStruct((N,), jnp.int32),
        ],
        mesh=mesh,
        scratch_types=[
            pltpu.VMEM((FC, D), jnp.float32),
            pltpu.VMEM((FC, D), jnp.float32),
            pltpu.VMEM((FC // 2, D), jnp.float32),
            pltpu.VMEM((FC // 2, D), jnp.float32),
            pltpu.VMEM((3, 2 * CROWS), jnp.float32),
            pltpu.VMEM((3, CROWS), jnp.float32),
            pltpu.VMEM((upt,), jnp.int32),
            pltpu.VMEM((_L,), jnp.int32),
            pltpu.SemaphoreType.DMA,
            pltpu.SemaphoreType.DMA,
            pltpu.SemaphoreType.DMA,
            pltpu.SemaphoreType.DMA,
        ],
    )
    def sc_kernel(feat_hbm, ct_hbm, seps_hbm,
                  out_feat, out_coord_t, out_sep, out_unpool,
                  fin0, fin1, fout0, fout1, colin, colout, ubuf, sbuf,
                  si0, si1, so0, so1):
        wid = lax.axis_index("s") * _NC + lax.axis_index("c")
        iota = lax.iota(jnp.int32, _L)

        row0 = wid * in_rows
        n_pairs = n_chunks // 2

        def r_in(g):
            r = row0 + jnp.minimum(g * FC, max_start)
            return pl.multiple_of(r, 16)

        def in_slice(g):
            return feat_hbm.at[pl.ds(r_in(g), FC), :]

        def out_slice(g):
            r2 = lax.shift_right_logical(r_in(g), 1)
            return out_feat.at[pl.ds(pl.multiple_of(r2, 8), FC // 2), :]

        def compute(fin, fout):
            @pl.loop(0, FC // 2)
            def _row(r):
                for q in range(D // _L):
                    a = fin[2 * r, pl.ds(q * _L, _L)]
                    b = fin[2 * r + 1, pl.ds(q * _L, _L)]
                    fout[r, pl.ds(q * _L, _L)] = jnp.maximum(a, b)

        pltpu.async_copy(in_slice(0), fin0, si0)
        pltpu.async_copy(in_slice(1), fin1, si1)

        @pl.loop(0, n_pairs)
        def _feat(gg):
            g0 = gg * 2
            for (b, fin, fout, si, so) in ((0, fin0, fout0, si0, so0),
                                           (1, fin1, fout1, si1, so1)):
                @pl.when(gg > 0)
                def _():
                    pltpu.make_async_copy(fout, out_slice(b), so).wait()

                pltpu.make_async_copy(in_slice(b), fin, si).wait()
                compute(fin, fout)
                pltpu.async_copy(fout, out_slice(g0 + b), so)

                @pl.when(gg + 1 < n_pairs)
                def _():
                    pltpu.async_copy(in_slice(g0 + b + 2), fin, si)

        pltpu.make_async_copy(fout0, out_slice(0), so0).wait()
        pltpu.make_async_copy(fout1, out_slice(1), so1).wait()

        orow0 = pl.multiple_of(jnp.minimum(wid * CROWS, R - CROWS), 128)

        def permute(vec, lane_idx):
            return lax.gather(
                vec, lane_idx[:, None],
                lax.GatherDimensionNumbers(offset_dims=(),
                                           collapsed_slice_dims=(0,),
                                           start_index_map=(0,)),
                (1,), mode=lax.GatherScatterMode.PROMISE_IN_BOUNDS)

        idx_e = jnp.bitwise_and(iota * 2, _L - 1)
        hi = iota >= (_L // 2)

        pltpu.sync_copy(ct_hbm.at[:, pl.ds(2 * orow0, 2 * CROWS)], colin)

        @pl.loop(0, CROWS // _L)
        def _col(v):
            for c in range(3):
                v0 = colin[c, pl.ds(2 * _L * v, _L)]
                v1 = colin[c, pl.ds(2 * _L * v + _L, _L)]
                a = jnp.where(hi, permute(v1, idx_e), permute(v0, idx_e))
                b = jnp.where(hi, permute(v1, idx_e + 1),
                              permute(v0, idx_e + 1))
                colout[c, pl.ds(_L * v, _L)] = (a + b) * 0.5

        pltpu.sync_copy(colout, out_coord_t.at[:, pl.ds(orow0, CROWS)])

        half = lax.shift_right_logical(iota, 1)
        u0 = wid * (upt // 2)

        @pl.loop(0, uvecs)
        def _unpool(v):
            ubuf[pl.ds(v * _L, _L)] = u0 + v * (_L // 2) + half

        pltpu.sync_copy(ubuf, out_unpool.at[pl.ds(wid * upt, upt)])

        @pl.when(wid == 0)
        def _sep():
            pltpu.sync_copy(seps_hbm, sbuf)
            sbuf[...] = lax.shift_right_logical(sbuf[...] + 1, 1)
            pltpu.sync_copy(sbuf, out_sep)

    rf, rct, rs, ui = sc_kernel(input_feat, coords.T, seps)
    return rf, rct.T, rs, ui
